# baseline (speedup 1.0000x reference)
import functools

import jax
import jax.numpy as jnp
from jax.experimental import pallas as pl
from jax.experimental.pallas import tpu as pltpu

D_MODEL = 512
HIDDEN = 256
NUM_CLUSTERS = 1024
EPS = 1e-6
U_MIN = 1e-4

_PREC = jax.lax.Precision.DEFAULT
_NCHUNK = 8


def _fused_kernel(x_ref, u_ref, m_ref, c_ref, wp_ref, bp_ref, wo_ref, bo_ref,
                  out_ref, xbuf_ref, cbuf_ref, sem_ref):
    L = u_ref.shape[1]
    CH = xbuf_ref.shape[1]

    ccopy = pltpu.make_async_copy(c_ref, cbuf_ref, sem_ref.at[_NCHUNK])
    ccopy.start()
    copies = [
        pltpu.make_async_copy(x_ref.at[pl.ds(k * CH, CH), :],
                              xbuf_ref.at[k], sem_ref.at[k])
        for k in range(_NCHUNK)
    ]
    for cp in copies:
        cp.start()

    wp = wp_ref[...]
    bp = bp_ref[...][None, :]
    wo = wo_ref[...]
    bo = bo_ref[...][None, :]

    ct = None
    h2 = None
    for k in range(_NCHUNK):
        row_start = k * CH
        b = row_start // L
        off = row_start % L

        copies[k].wait()
        x = xbuf_ref[k]

        h = jax.lax.dot_general(x, wp, (((1,), (0,)), ((), ())),
                                preferred_element_type=jnp.float32,
                                precision=_PREC)
        h = jax.nn.gelu(h + bp)
        ab = jax.lax.dot_general(h, wo, (((1,), (0,)), ((), ())),
                                 preferred_element_type=jnp.float32,
                                 precision=_PREC)
        ab = ab + bo
        abt = jnp.transpose(ab)
        alpha = jnp.clip(jax.nn.softplus(abt[0:1, :]) + 1.0, 1.0, 10.0)
        beta = jnp.clip(jax.nn.softplus(abt[1:2, :]) + 1.0, 1.0, 10.0)

        uc = jnp.clip(u_ref[pl.ds(b, 1), pl.ds(off, CH)], U_MIN, 1.0 - U_MIN)
        t = jnp.exp(jnp.log1p(-uc) / (beta + EPS))
        one_minus_t = jnp.clip(1.0 - t, EPS, 1.0)
        g = jnp.exp(jnp.log(one_minus_t) / (alpha + EPS))
        gates = jnp.clip(g, EPS, 1.0 - EPS)

        if k == 0:
            ccopy.wait()
            ct = jnp.transpose(cbuf_ref[...])
            h2 = 0.5 * jnp.sum(ct * ct, axis=0, keepdims=True)

        s = jax.lax.dot_general(x, ct, (((1,), (0,)), ((), ())),
                                preferred_element_type=jnp.float32,
                                precision=_PREC)
        e = s - h2
        emax = jnp.max(e, axis=1, keepdims=True)
        entity = (e[:, 0:1] >= emax).astype(jnp.float32)

        mrow = m_ref[pl.ds(b, 1), pl.ds(off, CH)]
        res = gates * jnp.transpose(entity)
        out_ref[pl.ds(b, 1), pl.ds(off, CH)] = res * mrow * mrow


@functools.partial(jax.jit, static_argnames=())
def kernel(embeddings, attention_mask, centroids, u, W_proj, b_proj, W_out,
           b_out):
    B, L, D = embeddings.shape
    N = B * L
    CH = N // _NCHUNK
    flat = embeddings.reshape(N, D)

    out = pl.pallas_call(
        _fused_kernel,
        grid=(1,),
        in_specs=[
            pl.BlockSpec(memory_space=pltpu.MemorySpace.HBM),
            pl.BlockSpec((B, L), lambda i: (0, 0)),
            pl.BlockSpec((B, L), lambda i: (0, 0)),
            pl.BlockSpec(memory_space=pltpu.MemorySpace.HBM),
            pl.BlockSpec((D, HIDDEN), lambda i: (0, 0)),
            pl.BlockSpec((HIDDEN,), lambda i: (0,)),
            pl.BlockSpec((HIDDEN, 2), lambda i: (0, 0)),
            pl.BlockSpec((2,), lambda i: (0,)),
        ],
        out_specs=pl.BlockSpec((B, L), lambda i: (0, 0)),
        out_shape=jax.ShapeDtypeStruct((B, L), jnp.float32),
        scratch_shapes=[pltpu.VMEM((_NCHUNK, CH, D), jnp.float32),
                        pltpu.VMEM((NUM_CLUSTERS, D), jnp.float32),
                        pltpu.SemaphoreType.DMA((_NCHUNK + 1,))],
    )(flat, u, attention_mask, centroids, W_proj, b_proj, W_out, b_out)
    return out


# speedup vs baseline: 1.0806x; 1.0806x over previous
import functools

import jax
import jax.numpy as jnp
from jax.experimental import pallas as pl
from jax.experimental.pallas import tpu as pltpu

D_MODEL = 512
HIDDEN = 256
NUM_CLUSTERS = 1024
EPS = 1e-6
U_MIN = 1e-4

_PREC = jax.lax.Precision.DEFAULT
_NCHUNK = 4


def _fused_kernel(x_ref, u_ref, m_ref, c_ref, wp_ref, bp_ref, wo_ref, bo_ref,
                  out_ref, xbuf_ref, cbuf_ref, sem_ref):
    L = u_ref.shape[1]
    CH = xbuf_ref.shape[1]

    ccopy = pltpu.make_async_copy(c_ref, cbuf_ref, sem_ref.at[_NCHUNK])
    ccopy.start()
    copies = [
        pltpu.make_async_copy(x_ref.at[pl.ds(k * CH, CH), :],
                              xbuf_ref.at[k], sem_ref.at[k])
        for k in range(_NCHUNK)
    ]
    for cp in copies:
        cp.start()

    wp = wp_ref[...]
    bp = bp_ref[...][None, :]
    wo = wo_ref[...]
    bo = bo_ref[...][None, :]

    ct = None
    h2 = None
    for k in range(_NCHUNK):
        row_start = k * CH
        b = row_start // L
        off = row_start % L

        copies[k].wait()
        x = xbuf_ref[k]

        h = jax.lax.dot_general(x, wp, (((1,), (0,)), ((), ())),
                                preferred_element_type=jnp.float32,
                                precision=_PREC)
        h = jax.nn.gelu(h + bp)
        ab = jax.lax.dot_general(h, wo, (((1,), (0,)), ((), ())),
                                 preferred_element_type=jnp.float32,
                                 precision=_PREC)
        ab = ab + bo
        abt = jnp.transpose(ab)
        alpha = jnp.clip(jax.nn.softplus(abt[0:1, :]) + 1.0, 1.0, 10.0)
        beta = jnp.clip(jax.nn.softplus(abt[1:2, :]) + 1.0, 1.0, 10.0)

        uc = jnp.clip(u_ref[pl.ds(b, 1), pl.ds(off, CH)], U_MIN, 1.0 - U_MIN)
        t = jnp.exp(jnp.log1p(-uc) / (beta + EPS))
        one_minus_t = jnp.clip(1.0 - t, EPS, 1.0)
        g = jnp.exp(jnp.log(one_minus_t) / (alpha + EPS))
        gates = jnp.clip(g, EPS, 1.0 - EPS)

        if k == 0:
            ccopy.wait()
            ct = jnp.transpose(cbuf_ref[...])
            h2 = 0.5 * jnp.sum(ct * ct, axis=0, keepdims=True)

        s = jax.lax.dot_general(x, ct, (((1,), (0,)), ((), ())),
                                preferred_element_type=jnp.float32,
                                precision=_PREC)
        e = s - h2
        emax = jnp.max(e, axis=1, keepdims=True)
        entity = (e[:, 0:1] >= emax).astype(jnp.float32)

        mrow = m_ref[pl.ds(b, 1), pl.ds(off, CH)]
        res = gates * jnp.transpose(entity)
        out_ref[pl.ds(b, 1), pl.ds(off, CH)] = res * mrow * mrow


@functools.partial(jax.jit, static_argnames=())
def kernel(embeddings, attention_mask, centroids, u, W_proj, b_proj, W_out,
           b_out):
    B, L, D = embeddings.shape
    N = B * L
    CH = N // _NCHUNK
    flat = embeddings.reshape(N, D)

    out = pl.pallas_call(
        _fused_kernel,
        grid=(1,),
        in_specs=[
            pl.BlockSpec(memory_space=pltpu.MemorySpace.HBM),
            pl.BlockSpec((B, L), lambda i: (0, 0)),
            pl.BlockSpec((B, L), lambda i: (0, 0)),
            pl.BlockSpec(memory_space=pltpu.MemorySpace.HBM),
            pl.BlockSpec((D, HIDDEN), lambda i: (0, 0)),
            pl.BlockSpec((HIDDEN,), lambda i: (0,)),
            pl.BlockSpec((HIDDEN, 2), lambda i: (0, 0)),
            pl.BlockSpec((2,), lambda i: (0,)),
        ],
        out_specs=pl.BlockSpec((B, L), lambda i: (0, 0)),
        out_shape=jax.ShapeDtypeStruct((B, L), jnp.float32),
        scratch_shapes=[pltpu.VMEM((_NCHUNK, CH, D), jnp.float32),
                        pltpu.VMEM((NUM_CLUSTERS, D), jnp.float32),
                        pltpu.SemaphoreType.DMA((_NCHUNK + 1,))],
    )(flat, u, attention_mask, centroids, W_proj, b_proj, W_out, b_out)
    return out


# speedup vs baseline: 1.1024x; 1.0201x over previous
import functools

import jax
import jax.numpy as jnp
from jax.experimental import pallas as pl
from jax.experimental.pallas import tpu as pltpu

D_MODEL = 512
HIDDEN = 256
NUM_CLUSTERS = 1024
EPS = 1e-6
U_MIN = 1e-4

_PREC = jax.lax.Precision.DEFAULT
_NCHUNK = 4


def _fused_kernel(x_ref, u_ref, m_ref, c_ref, wp_ref, bp_ref, wo_ref, bo_ref,
                  out_ref, xbuf_ref, sem_ref):
    L = u_ref.shape[1]

    copies = [
        pltpu.make_async_copy(x_ref.at[pl.ds(k * L, L), :],
                              xbuf_ref.at[k], sem_ref.at[k])
        for k in range(_NCHUNK)
    ]
    for cp in copies:
        cp.start()

    ct = jnp.transpose(c_ref[...])
    h2 = 0.5 * jnp.sum(ct * ct, axis=0, keepdims=True)

    wp = wp_ref[...]
    bp = bp_ref[...][None, :]
    wo = wo_ref[...]
    bo = bo_ref[...][None, :]

    for k in range(_NCHUNK):
        copies[k].wait()
        x = xbuf_ref[k]

        h = jax.lax.dot_general(x, wp, (((1,), (0,)), ((), ())),
                                preferred_element_type=jnp.float32,
                                precision=_PREC)
        h = jax.nn.gelu(h + bp)
        ab = jax.lax.dot_general(h, wo, (((1,), (0,)), ((), ())),
                                 preferred_element_type=jnp.float32,
                                 precision=_PREC)
        ab = ab + bo
        abt = jnp.transpose(ab)
        alpha = jnp.clip(jax.nn.softplus(abt[0:1, :]) + 1.0, 1.0, 10.0)
        beta = jnp.clip(jax.nn.softplus(abt[1:2, :]) + 1.0, 1.0, 10.0)

        uc = jnp.clip(u_ref[pl.ds(k, 1), :], U_MIN, 1.0 - U_MIN)
        t = jnp.exp(jnp.log1p(-uc) / (beta + EPS))
        one_minus_t = jnp.clip(1.0 - t, EPS, 1.0)
        g = jnp.exp(jnp.log(one_minus_t) / (alpha + EPS))
        gates = jnp.clip(g, EPS, 1.0 - EPS)

        s = jax.lax.dot_general(x, ct, (((1,), (0,)), ((), ())),
                                preferred_element_type=jnp.float32,
                                precision=_PREC)
        e = s - h2
        emax = jnp.max(e, axis=1, keepdims=True)
        entity = (e[:, 0:1] >= emax).astype(jnp.float32)

        mrow = m_ref[pl.ds(k, 1), :]
        res = gates * jnp.transpose(entity)
        out_ref[pl.ds(k, 1), :] = res * mrow * mrow


@functools.partial(jax.jit, static_argnames=())
def kernel(embeddings, attention_mask, centroids, u, W_proj, b_proj, W_out,
           b_out):
    B, L, D = embeddings.shape
    N = B * L
    flat = embeddings.reshape(N, D)

    out = pl.pallas_call(
        _fused_kernel,
        grid=(1,),
        in_specs=[
            pl.BlockSpec(memory_space=pltpu.MemorySpace.HBM),
            pl.BlockSpec((B, L), lambda i: (0, 0)),
            pl.BlockSpec((B, L), lambda i: (0, 0)),
            pl.BlockSpec((NUM_CLUSTERS, D), lambda i: (0, 0)),
            pl.BlockSpec((D, HIDDEN), lambda i: (0, 0)),
            pl.BlockSpec((HIDDEN,), lambda i: (0,)),
            pl.BlockSpec((HIDDEN, 2), lambda i: (0, 0)),
            pl.BlockSpec((2,), lambda i: (0,)),
        ],
        out_specs=pl.BlockSpec((B, L), lambda i: (0, 0)),
        out_shape=jax.ShapeDtypeStruct((B, L), jnp.float32),
        scratch_shapes=[pltpu.VMEM((_NCHUNK, L, D), jnp.float32),
                        pltpu.SemaphoreType.DMA((_NCHUNK,))],
    )(flat, u, attention_mask, centroids, W_proj, b_proj, W_out, b_out)
    return out


# speedup vs baseline: 1.1157x; 1.0121x over previous
import functools

import jax
import jax.numpy as jnp
from jax.experimental import pallas as pl
from jax.experimental.pallas import tpu as pltpu

D_MODEL = 512
HIDDEN = 256
NUM_CLUSTERS = 1024
EPS = 1e-6
U_MIN = 1e-4

_PREC = jax.lax.Precision.DEFAULT
_NCHUNK = 4


def _fused_kernel(x_ref, u_ref, m_ref, c_ref, wp_ref, bp_ref, wo_ref, bo_ref,
                  out_ref, xbuf_ref, cbuf_ref, sem_ref):
    L = u_ref.shape[1]

    copies = [
        pltpu.make_async_copy(x_ref.at[pl.ds(k * L, L), :],
                              xbuf_ref.at[k], sem_ref.at[k])
        for k in range(_NCHUNK)
    ]
    ccopy = pltpu.make_async_copy(c_ref, cbuf_ref, sem_ref.at[_NCHUNK])
    copies[0].start()
    ccopy.start()
    for cp in copies[1:]:
        cp.start()

    wp = wp_ref[...]
    bp = bp_ref[...][None, :]
    wo = wo_ref[...]
    bo = bo_ref[...][None, :]

    ct = None
    h2 = None
    for k in range(_NCHUNK):
        copies[k].wait()
        x = xbuf_ref[k]

        h = jax.lax.dot_general(x, wp, (((1,), (0,)), ((), ())),
                                preferred_element_type=jnp.float32,
                                precision=_PREC)
        h = jax.nn.gelu(h + bp)
        ab = jax.lax.dot_general(h, wo, (((1,), (0,)), ((), ())),
                                 preferred_element_type=jnp.float32,
                                 precision=_PREC)
        ab = ab + bo
        abt = jnp.transpose(ab)
        alpha = jnp.clip(jax.nn.softplus(abt[0:1, :]) + 1.0, 1.0, 10.0)
        beta = jnp.clip(jax.nn.softplus(abt[1:2, :]) + 1.0, 1.0, 10.0)

        uc = jnp.clip(u_ref[pl.ds(k, 1), :], U_MIN, 1.0 - U_MIN)
        t = jnp.exp(jnp.log1p(-uc) / (beta + EPS))
        one_minus_t = jnp.clip(1.0 - t, EPS, 1.0)
        g = jnp.exp(jnp.log(one_minus_t) / (alpha + EPS))
        gates = jnp.clip(g, EPS, 1.0 - EPS)

        if k == 0:
            ccopy.wait()
            ct = jnp.transpose(cbuf_ref[...])
            h2 = 0.5 * jnp.sum(ct * ct, axis=0, keepdims=True)

        s = jax.lax.dot_general(x, ct, (((1,), (0,)), ((), ())),
                                preferred_element_type=jnp.float32,
                                precision=_PREC)
        e = s - h2
        emax = jnp.max(e, axis=1, keepdims=True)
        entity = (e[:, 0:1] >= emax).astype(jnp.float32)

        mrow = m_ref[pl.ds(k, 1), :]
        res = gates * jnp.transpose(entity)
        out_ref[pl.ds(k, 1), :] = res * mrow * mrow


@functools.partial(jax.jit, static_argnames=())
def kernel(embeddings, attention_mask, centroids, u, W_proj, b_proj, W_out,
           b_out):
    B, L, D = embeddings.shape
    N = B * L
    flat = embeddings.reshape(N, D)

    out = pl.pallas_call(
        _fused_kernel,
        grid=(1,),
        in_specs=[
            pl.BlockSpec(memory_space=pltpu.MemorySpace.HBM),
            pl.BlockSpec((B, L), lambda i: (0, 0)),
            pl.BlockSpec((B, L), lambda i: (0, 0)),
            pl.BlockSpec(memory_space=pltpu.MemorySpace.HBM),
            pl.BlockSpec((D, HIDDEN), lambda i: (0, 0)),
            pl.BlockSpec((HIDDEN,), lambda i: (0,)),
            pl.BlockSpec((HIDDEN, 2), lambda i: (0, 0)),
            pl.BlockSpec((2,), lambda i: (0,)),
        ],
        out_specs=pl.BlockSpec((B, L), lambda i: (0, 0)),
        out_shape=jax.ShapeDtypeStruct((B, L), jnp.float32),
        scratch_shapes=[pltpu.VMEM((_NCHUNK, L, D), jnp.float32),
                        pltpu.VMEM((NUM_CLUSTERS, D), jnp.float32),
                        pltpu.SemaphoreType.DMA((_NCHUNK + 1,))],
    )(flat, u, attention_mask, centroids, W_proj, b_proj, W_out, b_out)
    return out
